# Initial kernel scaffold; baseline (speedup 1.0000x reference)
import jax
import jax.numpy as jnp
from jax import lax
from jax.experimental import pallas as pl
from jax.experimental.pallas import tpu as pltpu
from jax.experimental.pallas import tpu_sc as plsc

NTOKEN = 1000000
EMB_DIM = 64
BATCH = 16384
HIST_LEN = 50

NC = 2
NS = 16
NW = NC * NS

B = BATCH * HIST_LEN
PER_W = B // NW
CHUNK = 128
N_CHUNKS = PER_W // CHUNK
NBUF = 8
N_OUTER = N_CHUNKS // NBUF


def _body(table_hbm, idx_hbm, out_hbm, idx_v, rows, gsems, ssems):
  wid = lax.axis_index("s") * NC + lax.axis_index("c")
  base = wid * PER_W

  pltpu.sync_copy(idx_hbm.at[wid], idx_v)

  @pl.loop(0, N_OUTER)
  def _(g):
    j0 = g * NBUF
    descs = []
    for b in range(NBUF):
      descs.append(
          pltpu.async_copy(table_hbm.at[idx_v.at[j0 + b]], rows[b], gsems[b]))
    sdescs = []
    for b in range(NBUF):
      descs[b].wait()
      dst = out_hbm.at[pl.ds(base + (j0 + b) * CHUNK, CHUNK), :]
      sdescs.append(pltpu.async_copy(rows[b], dst, ssems[b]))
    for b in range(NBUF):
      sdescs[b].wait()


@jax.jit
def _lookup(x_flat3, emb_weight):
  mesh = plsc.VectorSubcoreMesh(
      core_axis_name="c", subcore_axis_name="s", num_cores=NC,
      num_subcores=NS)
  scratch = (
      [pltpu.VMEM((N_CHUNKS, CHUNK), jnp.int32)]
      + [[pltpu.VMEM((CHUNK, EMB_DIM), jnp.float32) for _ in range(NBUF)]]
      + [[pltpu.SemaphoreType.DMA for _ in range(NBUF)]]
      + [[pltpu.SemaphoreType.DMA for _ in range(NBUF)]]
  )
  return pl.kernel(
      _body,
      out_type=jax.ShapeDtypeStruct((B, EMB_DIM), jnp.float32),
      mesh=mesh,
      scratch_types=scratch,
  )(emb_weight, x_flat3)


def kernel(x, emb_weight):
  idx3 = x.astype(jnp.int32).reshape(NW, N_CHUNKS, CHUNK)
  out = _lookup(idx3, emb_weight)
  return out.reshape(BATCH, HIST_LEN, EMB_DIM)


# speedup vs baseline: 1.8836x; 1.8836x over previous
import jax
import jax.numpy as jnp
from jax import lax
from jax.experimental import pallas as pl
from jax.experimental.pallas import tpu as pltpu
from jax.experimental.pallas import tpu_sc as plsc

NTOKEN = 1000000
EMB_DIM = 64
BATCH = 16384
HIST_LEN = 50

NC = 2
NS = 16
NW = NC * NS

B = BATCH * HIST_LEN
PER_W = B // NW
CHUNK = 128
N_CHUNKS = PER_W // CHUNK
NBUF = 8
N_OUTER = N_CHUNKS // NBUF


def _body(table_hbm, idx_hbm, out_hbm, idx_v, rows, gsems, ssems):
  wid = lax.axis_index("s") * NC + lax.axis_index("c")
  base = wid * PER_W

  pltpu.sync_copy(idx_hbm.at[wid], idx_v)

  @pl.loop(0, N_OUTER)
  def _(g):
    j0 = g * NBUF
    descs = []
    for b in range(NBUF):
      descs.append(
          pltpu.async_copy(table_hbm.at[idx_v.at[j0 + b]], rows[b], gsems[b]))
    sdescs = []
    for b in range(NBUF):
      descs[b].wait()
      dst = out_hbm.at[pl.ds(base + (j0 + b) * CHUNK, CHUNK), :]
      sdescs.append(pltpu.async_copy(rows[b], dst, ssems[b]))
    for b in range(NBUF):
      sdescs[b].wait()


@jax.jit
def _lookup(x_flat3, emb_weight):
  mesh = plsc.VectorSubcoreMesh(
      core_axis_name="c", subcore_axis_name="s", num_cores=NC,
      num_subcores=NS)
  scratch = (
      [pltpu.VMEM((N_CHUNKS, CHUNK), jnp.int32)]
      + [[pltpu.VMEM((CHUNK, EMB_DIM), jnp.float32) for _ in range(NBUF)]]
      + [[pltpu.SemaphoreType.DMA for _ in range(NBUF)]]
      + [[pltpu.SemaphoreType.DMA for _ in range(NBUF)]]
  )
  return pl.kernel(
      _body,
      out_type=jax.ShapeDtypeStruct((B, EMB_DIM), jnp.float32),
      mesh=mesh,
      scratch_types=scratch,
      compiler_params=pltpu.CompilerParams(use_tc_tiling_on_sc=False),
  )(emb_weight, x_flat3)


def kernel(x, emb_weight):
  idx3 = x.astype(jnp.int32).reshape(NW, N_CHUNKS, CHUNK)
  out = _lookup(idx3, emb_weight)
  return out.reshape(BATCH, HIST_LEN, EMB_DIM)
